# speedup vs baseline: 1.6540x; 1.6540x over previous
import functools

import jax
import jax.numpy as jnp
from jax import lax
from jax.experimental import pallas as pl
from jax.experimental.pallas import tpu as pltpu
from jax.experimental.pallas import tpu_sc as plsc

D = 24576
B = 16 * 128
NC, NS = 2, 16
NW = NC * NS
BPW = B // NW
RPC = 4
NCH = BPW // RPC

_mesh = plsc.VectorSubcoreMesh(core_axis_name="c", subcore_axis_name="s")


@functools.partial(
    pl.kernel,
    mesh=_mesh,
    out_type=jax.ShapeDtypeStruct((B, D), jnp.float32),
    scratch_types=[
        pltpu.VMEM((NCH, RPC), jnp.int32),
        pltpu.VMEM((RPC, D), jnp.float32),
        pltpu.SemaphoreType.DMA,
    ],
)
def _gather(idx_hbm, table_hbm, out_hbm, idx_v, rows_v, sem):
    wid = lax.axis_index("s") * NC + lax.axis_index("c")
    pltpu.sync_copy(idx_hbm.at[wid], idx_v)
    base = wid * BPW

    def body(i, carry):
        pltpu.async_copy(table_hbm.at[idx_v.at[i]], rows_v, sem).wait()
        pltpu.sync_copy(rows_v, out_hbm.at[pl.ds(base + i * RPC, RPC)])
        return carry

    lax.fori_loop(0, NCH, body, 0)


def kernel(prefix, table):
    idx = prefix.reshape(NW, NCH, RPC).astype(jnp.int32)
    out = _gather(idx, table)
    return out.reshape(prefix.shape[0], prefix.shape[1], D)


# speedup vs baseline: 1.7288x; 1.0452x over previous
import functools

import jax
import jax.numpy as jnp
from jax import lax
from jax.experimental import pallas as pl
from jax.experimental.pallas import tpu as pltpu
from jax.experimental.pallas import tpu_sc as plsc

D = 24576
B = 16 * 128
NC, NS = 2, 16
NW = NC * NS
BPW = B // NW
RPC = 2
NCH = BPW // RPC

_mesh = plsc.VectorSubcoreMesh(core_axis_name="c", subcore_axis_name="s")


@functools.partial(
    pl.kernel,
    mesh=_mesh,
    out_type=jax.ShapeDtypeStruct((B, D), jnp.float32),
    scratch_types=[
        pltpu.VMEM((NCH, RPC), jnp.int32),
        pltpu.VMEM((RPC, D), jnp.float32),
        pltpu.VMEM((RPC, D), jnp.float32),
        pltpu.SemaphoreType.DMA,
        pltpu.SemaphoreType.DMA,
        pltpu.SemaphoreType.DMA,
        pltpu.SemaphoreType.DMA,
    ],
)
def _gather(idx_hbm, table_hbm, out_hbm, idx_v, buf0, buf1,
            gsem0, gsem1, ssem0, ssem1):
    wid = lax.axis_index("s") * NC + lax.axis_index("c")
    pltpu.sync_copy(idx_hbm.at[wid], idx_v)
    base = wid * BPW
    bufs = (buf0, buf1)
    gsems = (gsem0, gsem1)
    ssems = (ssem0, ssem1)

    def gather_start(i, b):
        pltpu.async_copy(table_hbm.at[idx_v.at[i]], bufs[b], gsems[b])

    def gather_wait(i, b):
        pltpu.make_async_copy(table_hbm.at[idx_v.at[i]], bufs[b],
                              gsems[b]).wait()

    def scatter_start(i, b):
        pltpu.async_copy(bufs[b], out_hbm.at[pl.ds(base + i * RPC, RPC)],
                         ssems[b])

    def scatter_wait(i, b):
        pltpu.make_async_copy(bufs[b], out_hbm.at[pl.ds(base + i * RPC, RPC)],
                              ssems[b]).wait()

    gather_start(0, 0)
    gather_start(1, 1)

    def body(j, carry):
        for b in range(2):
            i = 2 * j + b
            gather_wait(i, b)
            scatter_start(i, b)
            scatter_wait(i, b)
            gather_start(i + 2, b)
        return carry

    lax.fori_loop(0, NCH // 2 - 1, body, 0)

    for b in range(2):
        i = NCH - 2 + b
        gather_wait(i, b)
        scatter_start(i, b)
        scatter_wait(i, b)


def kernel(prefix, table):
    idx = prefix.reshape(NW, NCH, RPC).astype(jnp.int32)
    out = _gather(idx, table)
    return out.reshape(prefix.shape[0], prefix.shape[1], D)
